# baseline (speedup 1.0000x reference)
import functools

import jax
import jax.numpy as jnp
from jax import lax
from jax.experimental import pallas as pl
from jax.experimental.pallas import tpu as pltpu
from jax.experimental.pallas import tpu_sc as plsc

_LANES = 16
_CHUNK = 128


@functools.cache
def _build(B, V, D):
    info = plsc.get_sparse_core_info()
    nw = info.num_cores * info.num_subcores
    bpw = B // nw
    n_chunks = bpw // _CHUNK
    rec = 128 // D
    mesh = plsc.VectorSubcoreMesh(core_axis_name="c", subcore_axis_name="s")

    @functools.partial(
        pl.kernel,
        mesh=mesh,
        out_type=jax.ShapeDtypeStruct((B, D), jnp.float32),
        compiler_params=pltpu.CompilerParams(
            skip_device_barrier=True,
            disable_bounds_checks=True,
            disable_semaphore_checks=True,
        ),
        scratch_types=[
            pltpu.VMEM((n_chunks, _CHUNK), jnp.int32),
            pltpu.VMEM((bpw,), jnp.int32),
            pltpu.VMEM((bpw,), jnp.float32),
            pltpu.VMEM((bpw, 128), jnp.float32),
            pltpu.VMEM((2, _CHUNK, D), jnp.float32),
            [pltpu.SemaphoreType.DMA for _ in range(n_chunks)],
            [pltpu.SemaphoreType.DMA for _ in range(2)],
        ],
    )
    def gather_scale(x_hbm, gidx_hbm, half_hbm, sc_hbm, out_hbm,
                     gidx_v, half_v, sc_v, land_v, obuf_v, sems, osems):
        wid = lax.axis_index("s") * info.num_cores + lax.axis_index("c")
        base = wid * bpw
        xv = x_hbm
        for i in range(n_chunks):
            pltpu.sync_copy(gidx_hbm.at[pl.ds(base + i * _CHUNK, _CHUNK)],
                            gidx_v.at[i])
        pltpu.sync_copy(half_hbm.at[pl.ds(base, bpw)], half_v)
        pltpu.sync_copy(sc_hbm.at[pl.ds(base, bpw)], sc_v)

        gathers = [
            pltpu.async_copy(
                xv.at[gidx_v.at[i]],
                land_v.at[pl.ds(i * _CHUNK, _CHUNK)],
                sems[i],
            )
            for i in range(n_chunks)
        ]

        outs = []
        for i in range(n_chunks):
            slot = i % 2
            if i >= 2:
                outs[i - 2].wait()
            gathers[i].wait()

            def scale_group(t, carry):
                row0 = i * _CHUNK + t * _LANES
                s16 = sc_v[pl.ds(row0, _LANES)]
                h16 = half_v[pl.ds(row0, _LANES)]
                for r in range(_LANES):
                    s = s16[r]
                    h = h16[r]
                    off = h * D
                    for j in range(D // _LANES):
                        col = pl.ds(j * _LANES, _LANES)
                        src = pl.ds(off + j * _LANES, _LANES)
                        obuf_v[slot, t * _LANES + r, col] = (
                            land_v[row0 + r, src] * s)
                return carry

            lax.fori_loop(0, _CHUNK // _LANES, scale_group, 0)
            outs.append(pltpu.async_copy(
                obuf_v.at[slot],
                out_hbm.at[pl.ds(base + i * _CHUNK, _CHUNK)],
                osems[slot],
            ))
        for o in outs[-2:]:
            o.wait()

    return gather_scale


def kernel(label_idc, scores, X):
    B = label_idc.shape[0]
    V, D = X.shape
    idx = label_idc.astype(jnp.int32)
    gidx = lax.shift_right_logical(idx, 1)
    half = lax.bitwise_and(idx, 1)
    s = scores.reshape(B).astype(jnp.float32)
    x2 = X.reshape(V // 2, 2 * D)
    return _build(B, V, D)(x2, gidx, half, s)


# speedup vs baseline: 1.0860x; 1.0860x over previous
import functools

import jax
import jax.numpy as jnp
from jax import lax
from jax.experimental import pallas as pl
from jax.experimental.pallas import tpu as pltpu
from jax.experimental.pallas import tpu_sc as plsc

_LANES = 16
_CHUNK = 128


@functools.cache
def _build(B, V, D):
    info = plsc.get_sparse_core_info()
    nw = info.num_cores * info.num_subcores
    bpw = B // nw
    n_chunks = bpw // _CHUNK
    rec = 128 // D
    mesh = plsc.VectorSubcoreMesh(core_axis_name="c", subcore_axis_name="s")

    @functools.partial(
        pl.kernel,
        mesh=mesh,
        out_type=jax.ShapeDtypeStruct((B, D), jnp.float32),
        compiler_params=pltpu.CompilerParams(
            skip_device_barrier=True,
            disable_bounds_checks=True,
            disable_semaphore_checks=True,
        ),
        scratch_types=[
            pltpu.VMEM((n_chunks, _CHUNK), jnp.int32),
            pltpu.VMEM((bpw,), jnp.float32),
            pltpu.VMEM((bpw,), jnp.float32),
            pltpu.VMEM((bpw, 128), jnp.float32),
            pltpu.VMEM((2, _CHUNK, D), jnp.float32),
            [pltpu.SemaphoreType.DMA for _ in range(n_chunks)],
            [pltpu.SemaphoreType.DMA for _ in range(2)],
        ],
    )
    def gather_scale(x_hbm, gidx_hbm, s0_hbm, s1_hbm, out_hbm,
                     gidx_v, s0_v, s1_v, land_v, obuf_v, sems, osems):
        wid = lax.axis_index("s") * info.num_cores + lax.axis_index("c")
        base = wid * bpw
        xv = x_hbm
        for i in range(n_chunks):
            pltpu.sync_copy(gidx_hbm.at[pl.ds(base + i * _CHUNK, _CHUNK)],
                            gidx_v.at[i])
        pltpu.sync_copy(s0_hbm.at[pl.ds(base, bpw)], s0_v)
        pltpu.sync_copy(s1_hbm.at[pl.ds(base, bpw)], s1_v)

        gathers = [
            pltpu.async_copy(
                xv.at[gidx_v.at[i]],
                land_v.at[pl.ds(i * _CHUNK, _CHUNK)],
                sems[i],
            )
            for i in range(n_chunks)
        ]

        outs = []
        for i in range(n_chunks):
            slot = i % 2
            if i >= 2:
                outs[i - 2].wait()
            gathers[i].wait()

            def scale_group(t, carry):
                row0 = i * _CHUNK + t * _LANES
                a16 = s0_v[pl.ds(row0, _LANES)]
                b16 = s1_v[pl.ds(row0, _LANES)]
                for r in range(_LANES):
                    a = a16[r]
                    b = b16[r]
                    for j in range(D // _LANES):
                        col = pl.ds(j * _LANES, _LANES)
                        hi = pl.ds(D + j * _LANES, _LANES)
                        obuf_v[slot, t * _LANES + r, col] = (
                            land_v[row0 + r, col] * a
                            + land_v[row0 + r, hi] * b)
                return carry

            lax.fori_loop(0, _CHUNK // _LANES, scale_group, 0)
            outs.append(pltpu.async_copy(
                obuf_v.at[slot],
                out_hbm.at[pl.ds(base + i * _CHUNK, _CHUNK)],
                osems[slot],
            ))
        for o in outs[-2:]:
            o.wait()

    return gather_scale


def kernel(label_idc, scores, X):
    B = label_idc.shape[0]
    V, D = X.shape
    idx = label_idc.astype(jnp.int32)
    gidx = lax.shift_right_logical(idx, 1)
    half = lax.bitwise_and(idx, 1).astype(jnp.float32)
    s = scores.reshape(B).astype(jnp.float32)
    s1 = s * half
    s0 = s - s1
    x2 = X.reshape(V // 2, 2 * D)
    return _build(B, V, D)(x2, gidx, s0, s1)


# speedup vs baseline: 1.1644x; 1.0721x over previous
import functools

import jax
import jax.numpy as jnp
from jax import lax
from jax.experimental import pallas as pl
from jax.experimental.pallas import tpu as pltpu
from jax.experimental.pallas import tpu_sc as plsc

_LANES = 16
_TR = 8
_C = 32


@functools.cache
def _build(B, V, D):
    info = plsc.get_sparse_core_info()
    nw = info.num_cores * info.num_subcores
    bpw = B // nw
    n_chunks = bpw // _C
    n_steps = n_chunks // 2
    mesh = plsc.VectorSubcoreMesh(core_axis_name="c", subcore_axis_name="s")

    @functools.partial(
        pl.kernel,
        mesh=mesh,
        out_type=jax.ShapeDtypeStruct((B, D), jnp.float32),
        scratch_types=[
            pltpu.VMEM((bpw,), jnp.int32),
            pltpu.VMEM((bpw,), jnp.int32),
            pltpu.VMEM((bpw,), jnp.float32),
            pltpu.VMEM((_C, _TR, D), jnp.float32),
            pltpu.VMEM((_C, _TR, D), jnp.float32),
            pltpu.VMEM((_C // _TR, _TR, D), jnp.float32),
            pltpu.SemaphoreType.DMA,
            pltpu.SemaphoreType.DMA,
        ],
    )
    def gather_scale(x_hbm, tidx_hbm, ridx_hbm, sc_hbm, out_hbm,
                     tidx_v, ridx_v, sc_v, land0_v, land1_v, obuf_v,
                     sem0, sem1):
        wid = lax.axis_index("s") * info.num_cores + lax.axis_index("c")
        base = wid * bpw
        obase = wid * (bpw // _TR)
        xv = x_hbm.reshape(V // _TR, _TR, D)
        ov = out_hbm.reshape(B // _TR, _TR, D)
        pltpu.sync_copy(tidx_hbm.at[pl.ds(base, bpw)], tidx_v)
        pltpu.sync_copy(ridx_hbm.at[pl.ds(base, bpw)], ridx_v)
        pltpu.sync_copy(sc_hbm.at[pl.ds(base, bpw)], sc_v)

        def fire(chunk, land, sem):
            for i in range(_C // _LANES):
                t16 = tidx_v[pl.ds(chunk * _C + i * _LANES, _LANES)]
                for r in range(_LANES):
                    pltpu.async_copy(
                        xv.at[t16[r]], land.at[i * _LANES + r], sem)

        def drain(land, sem):
            pltpu.make_async_copy(xv.at[pl.ds(0, _C)], land, sem).wait()

        def process(chunk, land):
            for i in range(_C // _LANES):
                s16 = sc_v[pl.ds(chunk * _C + i * _LANES, _LANES)]
                r16 = ridx_v[pl.ds(chunk * _C + i * _LANES, _LANES)]
                for r in range(_LANES):
                    row = i * _LANES + r
                    rsel = r16[r]
                    s = s16[r]
                    for j in range(D // _LANES):
                        col = pl.ds(j * _LANES, _LANES)
                        obuf_v[row // _TR, row % _TR, col] = (
                            land[row, rsel, col] * s
                        )
            pltpu.sync_copy(
                obuf_v, ov.at[pl.ds(obase + chunk * (_C // _TR), _C // _TR)])

        fire(0, land0_v, sem0)

        def step(g, carry):
            fire(2 * g + 1, land1_v, sem1)
            drain(land0_v, sem0)
            process(2 * g, land0_v)

            @pl.when(g < n_steps - 1)
            def _():
                fire(2 * g + 2, land0_v, sem0)

            drain(land1_v, sem1)
            process(2 * g + 1, land1_v)
            return carry

        lax.fori_loop(0, n_steps, step, 0)

    return gather_scale


def kernel(label_idc, scores, X):
    B = label_idc.shape[0]
    V, D = X.shape
    idx = label_idc.astype(jnp.int32)
    tidx = lax.shift_right_logical(idx, 3)
    ridx = lax.bitwise_and(idx, 7)
    s = scores.reshape(B).astype(jnp.float32)
    return _build(B, V, D)(X, tidx, ridx, s)
